# baseline (speedup 1.0000x reference)
import math

import jax
import jax.numpy as jnp
from jax import lax
from jax.experimental import pallas as pl
from jax.experimental.pallas import tpu as pltpu
from jax.experimental.pallas import tpu_sc as plsc

N = 10000
D = 128
HD = D // 2
E = 320000
BN_SCALE = 1.0 / math.sqrt(1.0 + 1e-5)

NC = 2
NS = 16
NW = NC * NS

CHUNK = 128
ROWS = 10112
SEG = ROWS // NS

CH_D = 79
EPAD_D = NW * CH_D * CHUNK
CH_A = 160
EPAD_A = NS * CH_A * CHUNK

_f32 = jnp.float32
_mesh = plsc.VectorSubcoreMesh(core_axis_name="c", subcore_axis_name="s",
                               num_cores=NC, num_subcores=NS)


def _sc_degree(dst_hbm, zrow_hbm, onehot_hbm, deg_out, acc, ones_v, dst_v):
    c = lax.axis_index("c")
    s = lax.axis_index("s")
    w = c * NS + s

    pltpu.sync_copy(onehot_hbm, ones_v)
    pltpu.sync_copy(zrow_hbm, acc.at[pl.ds(s * SEG, SEG)])
    plsc.subcore_barrier()

    pltpu.sync_copy(dst_hbm.at[w], dst_v)

    def step(j, carry):
        pltpu.sync_copy(ones_v, acc.at[dst_v.at[j]], add=True)
        return carry

    lax.fori_loop(0, CH_D, step, 0)
    plsc.subcore_barrier()

    pltpu.sync_copy(acc.at[pl.ds(s * SEG, SEG)],
                    deg_out.at[c, pl.ds(s * SEG, SEG)])


def _sc_agg(y_hbm, src_hbm, dst_hbm, zrow_hbm, agg_out, acc, y_sp, src_v,
            dst_v, buf_v):
    c = lax.axis_index("c")
    s = lax.axis_index("s")

    pltpu.sync_copy(y_hbm.at[c, pl.ds(s * SEG, SEG)],
                    y_sp.at[pl.ds(s * SEG, SEG)])
    pltpu.sync_copy(zrow_hbm, acc.at[pl.ds(s * SEG, SEG)])
    plsc.subcore_barrier()

    pltpu.sync_copy(src_hbm.at[s], src_v)
    pltpu.sync_copy(dst_hbm.at[s], dst_v)

    def step(j, carry):
        pltpu.sync_copy(y_sp.at[src_v.at[j]], buf_v)
        pltpu.sync_copy(buf_v, acc.at[dst_v.at[j]], add=True)
        return carry

    lax.fori_loop(0, CH_A, step, 0, unroll=4)
    plsc.subcore_barrier()

    pltpu.sync_copy(acc.at[pl.ds(s * SEG, SEG)],
                    agg_out.at[c, pl.ds(s * SEG, SEG)])


_deg_kernel = pl.kernel(
    _sc_degree,
    out_type=jax.ShapeDtypeStruct((NC, ROWS, 16), _f32),
    mesh=_mesh,
    compiler_params=pltpu.CompilerParams(use_tc_tiling_on_sc=False),
    scratch_types=[
        pltpu.VMEM_SHARED((ROWS, 16), _f32),
        pltpu.VMEM((CHUNK, 16), _f32),
        pltpu.VMEM((CH_D, CHUNK), jnp.int32),
    ],
)

_agg_kernel = pl.kernel(
    _sc_agg,
    out_type=jax.ShapeDtypeStruct((NC, ROWS, HD), _f32),
    mesh=_mesh,
    compiler_params=pltpu.CompilerParams(use_tc_tiling_on_sc=False),
    scratch_types=[
        pltpu.VMEM_SHARED((ROWS, HD), _f32),
        pltpu.VMEM_SHARED((ROWS, HD), _f32),
        pltpu.VMEM((CH_A, CHUNK), jnp.int32),
        pltpu.VMEM((CH_A, CHUNK), jnp.int32),
        pltpu.VMEM((CHUNK, HD), _f32),
    ],
)


_BR = 2528
_GRID = ROWS // _BR


def _mm(a, w):
    return lax.dot_general(a, w, (((1,), (1,)), ((), ())),
                           preferred_element_type=_f32,
                           precision=lax.Precision.HIGHEST)


def _split(y, ref):
    ref[0] = y[:, :HD]
    ref[1] = y[:, HD:]


def _tc_mm1(x, w1, h1):
    h1[...] = _mm(x[...], w1[...])


def _tc_stage0(degp, h1, y1, dv):
    deg = degp[0, :, 0] + degp[1, :, 0] + 1.0
    dinv = lax.rsqrt(deg)[:, None]
    dv[...] = jnp.broadcast_to(dinv, (_BR, 16))
    _split(h1[...] * dinv, y1)


def _tc_mid(aggp, yp, dvb, w, b, g, be, ynext):
    dinv = dvb[:, 0:1]
    agg = jnp.concatenate([aggp[0] + yp[0], aggp[1] + yp[1]], axis=-1)
    conv = agg * dinv + b[...]
    z = jnp.maximum(g[...] * conv * BN_SCALE + be[...], 0.0)
    _split(_mm(z, w[...]) * dinv, ynext)


def _tc_final(aggp, yp, dvb, b, x, out):
    dinv = dvb[:, 0:1]
    agg = jnp.concatenate([aggp[0] + yp[0], aggp[1] + yp[1]], axis=-1)
    out[...] = agg * dinv + b[...] + x[...]


def _row_spec(bl=D):
    return pl.BlockSpec((_BR, bl), lambda i: (i, 0))


def _full_spec(shape):
    return pl.BlockSpec(shape, lambda i: tuple(0 for _ in shape))


def _slab_spec(bl=HD):
    return pl.BlockSpec((NC, _BR, bl), lambda i: (0, i, 0))


_y_shape = jax.ShapeDtypeStruct((NC, ROWS, HD), _f32)

_mm1 = pl.pallas_call(
    _tc_mm1,
    grid=(_GRID,),
    in_specs=[_row_spec(), _full_spec((D, D))],
    out_specs=_row_spec(),
    out_shape=jax.ShapeDtypeStruct((ROWS, D), _f32),
)

_stage0 = pl.pallas_call(
    _tc_stage0,
    grid=(_GRID,),
    in_specs=[_slab_spec(16), _row_spec()],
    out_specs=[_slab_spec(), _row_spec(16)],
    out_shape=[_y_shape, jax.ShapeDtypeStruct((ROWS, 16), _f32)],
)

_stage_mid = pl.pallas_call(
    _tc_mid,
    grid=(_GRID,),
    in_specs=[_slab_spec(), _slab_spec(), _row_spec(16), _full_spec((D, D)),
              _full_spec((1, D)), _full_spec((1, D)), _full_spec((1, D))],
    out_specs=_slab_spec(),
    out_shape=_y_shape,
)

_stage_final = pl.pallas_call(
    _tc_final,
    grid=(_GRID,),
    in_specs=[_slab_spec(), _slab_spec(), _row_spec(16), _full_spec((1, D)),
              _row_spec()],
    out_specs=_row_spec(),
    out_shape=jax.ShapeDtypeStruct((ROWS, D), _f32),
)


def _pad_edges(a, pad_val, nw, ch):
    pad = nw * ch * CHUNK - E
    return jnp.concatenate(
        [a, jnp.full((pad,), pad_val, jnp.int32)]).reshape(nw, ch, CHUNK)


@jax.jit
def kernel(x, edge_index, W1, b1, g1, be1, W2, b2, g2, be2, W3, b3):
    src = edge_index[0]
    dst = edge_index[1]
    dst_d = _pad_edges(dst, N, NW, CH_D)
    src_a = _pad_edges(src, 0, NS, CH_A)
    dst_a = _pad_edges(dst, N, NS, CH_A)
    x_p = jnp.concatenate([x, jnp.zeros((ROWS - N, D), _f32)], axis=0)

    b1r = b1.reshape(1, D)
    g1r = g1.reshape(1, D)
    be1r = be1.reshape(1, D)
    b2r = b2.reshape(1, D)
    g2r = g2.reshape(1, D)
    be2r = be2.reshape(1, D)
    b3r = b3.reshape(1, D)

    zrow = jnp.zeros((SEG, 16), _f32)
    zrow64 = jnp.zeros((SEG, HD), _f32)
    onehot = jnp.zeros((CHUNK, 16), _f32).at[:, 0].set(1.0)
    h1 = _mm1(x_p, W1)
    degp = _deg_kernel(dst_d, zrow, onehot)
    y1, dvb = _stage0(degp, h1)
    p1 = _agg_kernel(y1, src_a, dst_a, zrow64)
    y2 = _stage_mid(p1, y1, dvb, W2, b1r, g1r, be1r)
    p2 = _agg_kernel(y2, src_a, dst_a, zrow64)
    y3 = _stage_mid(p2, y2, dvb, W3, b2r, g2r, be2r)
    p3 = _agg_kernel(y3, src_a, dst_a, zrow64)
    out = _stage_final(p3, y3, dvb, b3r, x_p)
    return out[:N]


# speedup vs baseline: 1.0005x; 1.0005x over previous
import math

import jax
import jax.numpy as jnp
from jax import lax
from jax.experimental import pallas as pl
from jax.experimental.pallas import tpu as pltpu
from jax.experimental.pallas import tpu_sc as plsc

N = 10000
D = 128
HD = D // 2
E = 320000
BN_SCALE = 1.0 / math.sqrt(1.0 + 1e-5)

NC = 2
NS = 16
NW = NC * NS

CHUNK = 128
ROWS = 10112
SEG = ROWS // NS

CH_D = 79
EPAD_D = NW * CH_D * CHUNK
CH_A = 160
EPAD_A = NS * CH_A * CHUNK

_f32 = jnp.float32
_mesh = plsc.VectorSubcoreMesh(core_axis_name="c", subcore_axis_name="s",
                               num_cores=NC, num_subcores=NS)


def _sc_degree(dst_hbm, zrow_hbm, onehot_hbm, deg_out, acc, ones_v, dst_v):
    c = lax.axis_index("c")
    s = lax.axis_index("s")
    w = c * NS + s

    pltpu.sync_copy(onehot_hbm, ones_v)
    pltpu.sync_copy(zrow_hbm, acc.at[pl.ds(s * SEG, SEG)])
    plsc.subcore_barrier()

    pltpu.sync_copy(dst_hbm.at[w], dst_v)

    def step(j, carry):
        pltpu.sync_copy(ones_v, acc.at[dst_v.at[j]], add=True)
        return carry

    lax.fori_loop(0, CH_D, step, 0)
    plsc.subcore_barrier()

    pltpu.sync_copy(acc.at[pl.ds(s * SEG, SEG)],
                    deg_out.at[c, pl.ds(s * SEG, SEG)])


def _sc_agg(y_hbm, src_hbm, dst_hbm, zrow_hbm, agg_out, acc, y_sp, src_v,
            dst_v, buf_v):
    c = lax.axis_index("c")
    s = lax.axis_index("s")

    pltpu.sync_copy(y_hbm.at[c, pl.ds(s * SEG, SEG)],
                    y_sp.at[pl.ds(s * SEG, SEG)])
    pltpu.sync_copy(zrow_hbm, acc.at[pl.ds(s * SEG, SEG)])
    plsc.subcore_barrier()

    pltpu.sync_copy(src_hbm.at[s], src_v)
    pltpu.sync_copy(dst_hbm.at[s], dst_v)

    def step(j, carry):
        pltpu.sync_copy(y_sp.at[src_v.at[j]], buf_v)
        pltpu.sync_copy(buf_v, acc.at[dst_v.at[j]], add=True)
        return carry

    lax.fori_loop(0, CH_A, step, 0)
    plsc.subcore_barrier()

    pltpu.sync_copy(acc.at[pl.ds(s * SEG, SEG)],
                    agg_out.at[c, pl.ds(s * SEG, SEG)])


_deg_kernel = pl.kernel(
    _sc_degree,
    out_type=jax.ShapeDtypeStruct((NC, ROWS, 16), _f32),
    mesh=_mesh,
    compiler_params=pltpu.CompilerParams(use_tc_tiling_on_sc=False),
    scratch_types=[
        pltpu.VMEM_SHARED((ROWS, 16), _f32),
        pltpu.VMEM((CHUNK, 16), _f32),
        pltpu.VMEM((CH_D, CHUNK), jnp.int32),
    ],
)

_agg_kernel = pl.kernel(
    _sc_agg,
    out_type=jax.ShapeDtypeStruct((NC, ROWS, HD), _f32),
    mesh=_mesh,
    compiler_params=pltpu.CompilerParams(use_tc_tiling_on_sc=False),
    scratch_types=[
        pltpu.VMEM_SHARED((ROWS, HD), _f32),
        pltpu.VMEM_SHARED((ROWS, HD), _f32),
        pltpu.VMEM((CH_A, CHUNK), jnp.int32),
        pltpu.VMEM((CH_A, CHUNK), jnp.int32),
        pltpu.VMEM((CHUNK, HD), _f32),
    ],
)


_BR = 2528
_GRID = ROWS // _BR


def _mm(a, w):
    return lax.dot_general(a, w, (((1,), (1,)), ((), ())),
                           preferred_element_type=_f32,
                           precision=lax.Precision.HIGHEST)


def _split(y, ref):
    ref[0] = y[:, :HD]
    ref[1] = y[:, HD:]


def _tc_stage0(degp, x, w1, y1, dv):
    deg = degp[0, :, 0] + degp[1, :, 0] + 1.0
    dinv = lax.rsqrt(deg)[:, None]
    dv[...] = jnp.broadcast_to(dinv, (_BR, 16))
    _split(_mm(x[...], w1[...]) * dinv, y1)


def _tc_mid(aggp, yp, dvb, w, b, g, be, ynext):
    dinv = dvb[:, 0:1]
    agg = jnp.concatenate([aggp[0] + yp[0], aggp[1] + yp[1]], axis=-1)
    conv = agg * dinv + b[...]
    z = jnp.maximum(g[...] * conv * BN_SCALE + be[...], 0.0)
    _split(_mm(z, w[...]) * dinv, ynext)


def _tc_final(aggp, yp, dvb, b, x, out):
    dinv = dvb[:, 0:1]
    agg = jnp.concatenate([aggp[0] + yp[0], aggp[1] + yp[1]], axis=-1)
    out[...] = agg * dinv + b[...] + x[...]


def _row_spec(bl=D):
    return pl.BlockSpec((_BR, bl), lambda i: (i, 0))


def _full_spec(shape):
    return pl.BlockSpec(shape, lambda i: tuple(0 for _ in shape))


def _slab_spec(bl=HD):
    return pl.BlockSpec((NC, _BR, bl), lambda i: (0, i, 0))


_y_shape = jax.ShapeDtypeStruct((NC, ROWS, HD), _f32)

_stage0 = pl.pallas_call(
    _tc_stage0,
    grid=(_GRID,),
    in_specs=[_slab_spec(16), _row_spec(), _full_spec((D, D))],
    out_specs=[_slab_spec(), _row_spec(16)],
    out_shape=[_y_shape, jax.ShapeDtypeStruct((ROWS, 16), _f32)],
)

_stage_mid = pl.pallas_call(
    _tc_mid,
    grid=(_GRID,),
    in_specs=[_slab_spec(), _slab_spec(), _row_spec(16), _full_spec((D, D)),
              _full_spec((1, D)), _full_spec((1, D)), _full_spec((1, D))],
    out_specs=_slab_spec(),
    out_shape=_y_shape,
)

_stage_final = pl.pallas_call(
    _tc_final,
    grid=(_GRID,),
    in_specs=[_slab_spec(), _slab_spec(), _row_spec(16), _full_spec((1, D)),
              _row_spec()],
    out_specs=_row_spec(),
    out_shape=jax.ShapeDtypeStruct((ROWS, D), _f32),
)


def _pad_edges(a, pad_val, nw, ch):
    pad = nw * ch * CHUNK - E
    return jnp.concatenate(
        [a, jnp.full((pad,), pad_val, jnp.int32)]).reshape(nw, ch, CHUNK)


@jax.jit
def kernel(x, edge_index, W1, b1, g1, be1, W2, b2, g2, be2, W3, b3):
    src = edge_index[0]
    dst = edge_index[1]
    dst_d = _pad_edges(dst, N, NW, CH_D)
    src_a = _pad_edges(src, 0, NS, CH_A)
    dst_a = _pad_edges(dst, N, NS, CH_A)
    x_p = jnp.concatenate([x, jnp.zeros((ROWS - N, D), _f32)], axis=0)

    b1r = b1.reshape(1, D)
    g1r = g1.reshape(1, D)
    be1r = be1.reshape(1, D)
    b2r = b2.reshape(1, D)
    g2r = g2.reshape(1, D)
    be2r = be2.reshape(1, D)
    b3r = b3.reshape(1, D)

    zrow = jnp.zeros((SEG, 16), _f32)
    zrow64 = jnp.zeros((SEG, HD), _f32)
    onehot = jnp.zeros((CHUNK, 16), _f32).at[:, 0].set(1.0)
    degp = _deg_kernel(dst_d, zrow, onehot)
    y1, dvb = _stage0(degp, x_p, W1)
    p1 = _agg_kernel(y1, src_a, dst_a, zrow64)
    y2 = _stage_mid(p1, y1, dvb, W2, b1r, g1r, be1r)
    p2 = _agg_kernel(y2, src_a, dst_a, zrow64)
    y3 = _stage_mid(p2, y2, dvb, W3, b2r, g2r, be2r)
    p3 = _agg_kernel(y3, src_a, dst_a, zrow64)
    out = _stage_final(p3, y3, dvb, b3r, x_p)
    return out[:N]


# speedup vs baseline: 1.0208x; 1.0203x over previous
import math

import jax
import jax.numpy as jnp
from jax import lax
from jax.experimental import pallas as pl
from jax.experimental.pallas import tpu as pltpu
from jax.experimental.pallas import tpu_sc as plsc

N = 10000
D = 128
HD = D // 2
E = 320000
BN_SCALE = 1.0 / math.sqrt(1.0 + 1e-5)

NC = 2
NS = 16
NW = NC * NS

CHUNK = 128
ROWS = 10112
SEG = ROWS // NS

CH_D = 79
EPAD_D = NW * CH_D * CHUNK
CH_A = 157
EPAD_A = NS * CH_A * CHUNK

_f32 = jnp.float32
_mesh = plsc.VectorSubcoreMesh(core_axis_name="c", subcore_axis_name="s",
                               num_cores=NC, num_subcores=NS)


def _sc_degree(dst_hbm, zrow_hbm, onehot_hbm, deg_out, acc, ones_v, dst_v):
    c = lax.axis_index("c")
    s = lax.axis_index("s")
    w = c * NS + s

    pltpu.sync_copy(onehot_hbm, ones_v)
    pltpu.sync_copy(zrow_hbm, acc.at[pl.ds(s * SEG, SEG)])
    plsc.subcore_barrier()

    pltpu.sync_copy(dst_hbm.at[w], dst_v)

    def step(j, carry):
        pltpu.sync_copy(ones_v, acc.at[dst_v.at[j]], add=True)
        return carry

    lax.fori_loop(0, CH_D, step, 0)
    plsc.subcore_barrier()

    pltpu.sync_copy(acc.at[pl.ds(s * SEG, SEG)],
                    deg_out.at[c, pl.ds(s * SEG, SEG)])


def _sc_agg(y_hbm, src_hbm, dst_hbm, zrow_hbm, agg_out, acc, y_sp, src_v,
            dst_v, buf_v):
    c = lax.axis_index("c")
    s = lax.axis_index("s")

    pltpu.sync_copy(y_hbm.at[c, pl.ds(s * SEG, SEG)],
                    y_sp.at[pl.ds(s * SEG, SEG)])
    pltpu.sync_copy(zrow_hbm, acc.at[pl.ds(s * SEG, SEG)])
    plsc.subcore_barrier()

    pltpu.sync_copy(src_hbm.at[s], src_v)
    pltpu.sync_copy(dst_hbm.at[s], dst_v)

    def step(j, carry):
        pltpu.sync_copy(y_sp.at[src_v.at[j]], buf_v)
        pltpu.sync_copy(buf_v, acc.at[dst_v.at[j]], add=True)
        return carry

    lax.fori_loop(0, CH_A, step, 0)
    plsc.subcore_barrier()

    pltpu.sync_copy(acc.at[pl.ds(s * SEG, SEG)],
                    agg_out.at[c, pl.ds(s * SEG, SEG)])


_deg_kernel = pl.kernel(
    _sc_degree,
    out_type=jax.ShapeDtypeStruct((NC, ROWS, 16), _f32),
    mesh=_mesh,
    compiler_params=pltpu.CompilerParams(use_tc_tiling_on_sc=False),
    scratch_types=[
        pltpu.VMEM_SHARED((ROWS, 16), _f32),
        pltpu.VMEM((CHUNK, 16), _f32),
        pltpu.VMEM((CH_D, CHUNK), jnp.int32),
    ],
)

_agg_kernel = pl.kernel(
    _sc_agg,
    out_type=jax.ShapeDtypeStruct((NC, ROWS, HD), _f32),
    mesh=_mesh,
    compiler_params=pltpu.CompilerParams(use_tc_tiling_on_sc=False),
    scratch_types=[
        pltpu.VMEM_SHARED((ROWS, HD), _f32),
        pltpu.VMEM_SHARED((ROWS, HD), _f32),
        pltpu.VMEM((CH_A, CHUNK), jnp.int32),
        pltpu.VMEM((CH_A, CHUNK), jnp.int32),
        pltpu.VMEM((CHUNK, HD), _f32),
    ],
)


_BR = 2528
_GRID = ROWS // _BR


def _mm(a, w):
    return lax.dot_general(a, w, (((1,), (1,)), ((), ())),
                           preferred_element_type=_f32,
                           precision=lax.Precision.HIGHEST)


def _split(y, ref):
    ref[0] = y[:, :HD]
    ref[1] = y[:, HD:]


def _tc_stage0(degp, x, w1, y1, dv):
    deg = degp[0, :, 0] + degp[1, :, 0] + 1.0
    dinv = lax.rsqrt(deg)[:, None]
    dv[...] = jnp.broadcast_to(dinv, (_BR, 16))
    _split(_mm(x[...], w1[...]) * dinv, y1)


def _tc_mid(aggp, yp, dvb, w, b, g, be, ynext):
    dinv = dvb[:, 0:1]
    agg = jnp.concatenate([aggp[0] + yp[0], aggp[1] + yp[1]], axis=-1)
    conv = agg * dinv + b[...]
    z = jnp.maximum(g[...] * conv * BN_SCALE + be[...], 0.0)
    _split(_mm(z, w[...]) * dinv, ynext)


def _tc_final(aggp, yp, dvb, b, x, out):
    dinv = dvb[:, 0:1]
    agg = jnp.concatenate([aggp[0] + yp[0], aggp[1] + yp[1]], axis=-1)
    out[...] = agg * dinv + b[...] + x[...]


def _row_spec(bl=D):
    return pl.BlockSpec((_BR, bl), lambda i: (i, 0))


def _full_spec(shape):
    return pl.BlockSpec(shape, lambda i: tuple(0 for _ in shape))


def _slab_spec(bl=HD):
    return pl.BlockSpec((NC, _BR, bl), lambda i: (0, i, 0))


_y_shape = jax.ShapeDtypeStruct((NC, ROWS, HD), _f32)

_stage0 = pl.pallas_call(
    _tc_stage0,
    grid=(_GRID,),
    in_specs=[_slab_spec(16), _row_spec(), _full_spec((D, D))],
    out_specs=[_slab_spec(), _row_spec(16)],
    out_shape=[_y_shape, jax.ShapeDtypeStruct((ROWS, 16), _f32)],
)

_stage_mid = pl.pallas_call(
    _tc_mid,
    grid=(_GRID,),
    in_specs=[_slab_spec(), _slab_spec(), _row_spec(16), _full_spec((D, D)),
              _full_spec((1, D)), _full_spec((1, D)), _full_spec((1, D))],
    out_specs=_slab_spec(),
    out_shape=_y_shape,
)

_stage_final = pl.pallas_call(
    _tc_final,
    grid=(_GRID,),
    in_specs=[_slab_spec(), _slab_spec(), _row_spec(16), _full_spec((1, D)),
              _row_spec()],
    out_specs=_row_spec(),
    out_shape=jax.ShapeDtypeStruct((ROWS, D), _f32),
)


def _pad_edges(a, pad_val, nw, ch):
    pad = nw * ch * CHUNK - E
    return jnp.concatenate(
        [a, jnp.full((pad,), pad_val, jnp.int32)]).reshape(nw, ch, CHUNK)


@jax.jit
def kernel(x, edge_index, W1, b1, g1, be1, W2, b2, g2, be2, W3, b3):
    src = edge_index[0]
    dst = edge_index[1]
    dst_d = _pad_edges(dst, N, NW, CH_D)
    src_a = _pad_edges(src, 0, NS, CH_A)
    dst_a = _pad_edges(dst, N, NS, CH_A)
    x_p = jnp.concatenate([x, jnp.zeros((ROWS - N, D), _f32)], axis=0)

    b1r = b1.reshape(1, D)
    g1r = g1.reshape(1, D)
    be1r = be1.reshape(1, D)
    b2r = b2.reshape(1, D)
    g2r = g2.reshape(1, D)
    be2r = be2.reshape(1, D)
    b3r = b3.reshape(1, D)

    zrow = jnp.zeros((SEG, 16), _f32)
    zrow64 = jnp.zeros((SEG, HD), _f32)
    onehot = jnp.zeros((CHUNK, 16), _f32).at[:, 0].set(1.0)
    degp = _deg_kernel(dst_d, zrow, onehot)
    y1, dvb = _stage0(degp, x_p, W1)
    p1 = _agg_kernel(y1, src_a, dst_a, zrow64)
    y2 = _stage_mid(p1, y1, dvb, W2, b1r, g1r, be1r)
    p2 = _agg_kernel(y2, src_a, dst_a, zrow64)
    y3 = _stage_mid(p2, y2, dvb, W3, b2r, g2r, be2r)
    p3 = _agg_kernel(y3, src_a, dst_a, zrow64)
    out = _stage_final(p3, y3, dvb, b3r, x_p)
    return out[:N]
